# baseline (speedup 1.0000x reference)
import functools

import jax
import jax.numpy as jnp
from jax import lax
from jax.experimental import pallas as pl
from jax.experimental.pallas import tpu as pltpu
from jax.experimental.pallas import tpu_sc as plsc

B = 16384
D = 64
HID = 16

_info = plsc.get_sparse_core_info()
NC, NS = _info.num_cores, _info.num_subcores
NW = NC * NS
B_PER_W = B // NW
CHUNK = 128
NCH = B_PER_W // CHUNK
L = 16

_mesh = plsc.VectorSubcoreMesh(core_axis_name="c", subcore_axis_name="s")


@functools.partial(
    pl.kernel,
    mesh=_mesh,
    out_type=jax.ShapeDtypeStruct((D, B), jnp.float32),
    scratch_types=[
        pltpu.VMEM((NCH, CHUNK), jnp.int32),
        pltpu.VMEM((B_PER_W, D), jnp.float32),
        pltpu.VMEM((D, B_PER_W), jnp.float32),
        pltpu.SemaphoreType.DMA,
    ],
    compiler_params=pltpu.CompilerParams(
        use_tc_tiling_on_sc=False, needs_layout_passes=False
    ),
)
def _sc_gather_t(idx_hbm, table_hbm, emb_t_hbm, idx_v, rows_v, out_v, sem):
    wid = lax.axis_index("s") * NC + lax.axis_index("c")
    jbase = wid * B_PER_W
    pltpu.sync_copy(idx_hbm.at[wid], idx_v)
    handles = []
    for k in range(NCH):
        handles.append(
            pltpu.async_copy(
                table_hbm.at[idx_v.at[k]],
                rows_v.at[pl.ds(k * CHUNK, CHUNK)],
                sem,
            )
        )
    for h in handles:
        h.wait()

    def body(jg, _):
        jv = jg * L + lax.iota(jnp.int32, L)
        for c in range(D):
            cv = jnp.full((L,), c, dtype=jnp.int32)
            val = plsc.load_gather(rows_v, [jv, cv])
            out_v[c, pl.ds(jg * L, L)] = val
        return 0

    lax.fori_loop(0, B_PER_W // L, body, 0)
    pltpu.sync_copy(out_v, emb_t_hbm.at[:, pl.ds(jbase, B_PER_W)])


def _mlp_body(embt_ref, w1t_ref, b1_ref, w2t_ref, b2_ref, val_ref, embt_out_ref):
    embt = embt_ref[...]
    embt_out_ref[...] = embt
    h = jnp.dot(w1t_ref[...], embt, preferred_element_type=jnp.float32)
    h = jnp.maximum(h + b1_ref[...], 0.0)
    val_ref[...] = (
        jnp.dot(w2t_ref[...], h, preferred_element_type=jnp.float32) + b2_ref[...]
    )


_BJ = 2048


def _tc_mlp_t(embt, w1t, b1, w2t, b2):
    grid = (B // _BJ,)
    return pl.pallas_call(
        _mlp_body,
        grid=grid,
        in_specs=[
            pl.BlockSpec((D, _BJ), lambda j: (0, j)),
            pl.BlockSpec((HID, D), lambda j: (0, 0)),
            pl.BlockSpec((HID, 1), lambda j: (0, 0)),
            pl.BlockSpec((1, HID), lambda j: (0, 0)),
            pl.BlockSpec((1, 1), lambda j: (0, 0)),
        ],
        out_specs=[
            pl.BlockSpec((1, _BJ), lambda j: (0, j)),
            pl.BlockSpec((D, _BJ), lambda j: (0, j)),
        ],
        out_shape=[
            jax.ShapeDtypeStruct((1, B), jnp.float32),
            jax.ShapeDtypeStruct((D, B), jnp.float32),
        ],
    )(embt, w1t, b1, w2t, b2)


def kernel(states, emb_table, W1, b1, W2, b2):
    idx = states.reshape(NW, NCH, CHUNK)
    emb_t = _sc_gather_t(idx, emb_table)
    values_t, emb_t_tiled = _tc_mlp_t(
        emb_t, W1.T, b1.reshape(HID, 1), W2.T, b2.reshape(1, 1)
    )
    return emb_t_tiled.T, values_t.reshape(B, 1)
